# baseline (speedup 1.0000x reference)
import functools

import jax
import jax.numpy as jnp
from jax import lax
from jax.experimental import pallas as pl
from jax.experimental.pallas import tpu as pltpu
from jax.experimental.pallas import tpu_sc as plsc

NC = 2
NS = 16
L = 16
NW = NC * NS

CH = 128


def _sc_scatter(n_nodes, d, ep, acc_rows):
    e_per_tile = ep // NW
    n_ch = e_per_tile // CH
    nbuf = 2
    assert n_ch % nbuf == 0
    rows_per_tile = acc_rows // NS
    n_zero = rows_per_tile // CH

    mesh = plsc.VectorSubcoreMesh(core_axis_name="c", subcore_axis_name="s")

    @functools.partial(
        pl.kernel,
        mesh=mesh,
        out_type=jax.ShapeDtypeStruct((NC, acc_rows, d), jnp.float32),
        scratch_types=(
            [pltpu.VMEM((2, CH), jnp.int32) for _ in range(nbuf)]
            + [pltpu.VMEM((CH, d), jnp.float32) for _ in range(nbuf)]
            + [pltpu.VMEM_SHARED((acc_rows, d), jnp.float32)]
            + [pltpu.SemaphoreType.DMA for _ in range(3 * nbuf)]
        ),
    )
    def k(feat_hbm, edge_hbm, out_hbm, *scr):
        idx = scr[0:nbuf]
        rows = scr[nbuf:2 * nbuf]
        acc_sh = scr[2 * nbuf]
        sem_i = scr[2 * nbuf + 1:2 * nbuf + 1 + nbuf]
        sem_g = scr[2 * nbuf + 1 + nbuf:2 * nbuf + 1 + 2 * nbuf]
        sem_s = scr[2 * nbuf + 1 + 2 * nbuf:]
        cid = lax.axis_index("c")
        sid = lax.axis_index("s")
        wid = sid * NC + cid

        def zero_body(i, _):
            for j in range(d // L):
                rows[0][i, pl.ds(j * L, L)] = jnp.zeros((L,), jnp.float32)
            return _
        lax.fori_loop(0, CH, zero_body, None)
        acc_base = sid * rows_per_tile
        for j in range(n_zero):
            pltpu.sync_copy(rows[0], acc_sh.at[pl.ds(acc_base + j * CH, CH)])
        plsc.subcore_barrier()

        def start_gather(b):
            pltpu.async_copy(feat_hbm.at[idx[b].at[0]], rows[b], sem_g[b])

        def wait_gather(b):
            pltpu.make_async_copy(
                feat_hbm.at[idx[b].at[0]], rows[b], sem_g[b]).wait()

        def start_scatter(b):
            pltpu.async_copy(
                rows[b], acc_sh.at[idx[b].at[1]], sem_s[b], add=True)

        def wait_scatter(b):
            pltpu.make_async_copy(
                rows[b], acc_sh.at[idx[b].at[1]], sem_s[b]).wait()

        def start_idx(c, b):
            pltpu.async_copy(edge_hbm.at[wid, c], idx[b], sem_i[b])

        def wait_idx(c, b):
            pltpu.make_async_copy(edge_hbm.at[wid, c], idx[b], sem_i[b]).wait()

        def edge_body(c, _):
            pltpu.sync_copy(edge_hbm.at[wid, c], idx[0])
            pltpu.sync_copy(feat_hbm.at[idx[0].at[0]], rows[0])
            pltpu.sync_copy(rows[0], acc_sh.at[idx[0].at[1]], add=True)
            return _
        lax.fori_loop(0, n_ch, edge_body, None)
        plsc.subcore_barrier()

        pltpu.sync_copy(
            acc_sh.at[pl.ds(acc_base, rows_per_tile)],
            out_hbm.at[cid, pl.ds(acc_base, rows_per_tile)],
        )

    return k


def _combine_body(p0_ref, p1_ref, w_ref, o_ref):
    o_ref[...] = (p0_ref[0] + p1_ref[0]) * w_ref[...]


def kernel(features, edge_index, W):
    n_nodes, d = features.shape
    e = edge_index.shape[1]

    ep = -(-e // (4 * NW * CH)) * (4 * NW * CH)
    ei = edge_index
    if ep != e:
        pad = ep - e
        ei = jnp.concatenate(
            [ei, jnp.stack([jnp.zeros((pad,), jnp.int32),
                            jnp.full((pad,), n_nodes, jnp.int32)])], axis=1)
    n_ch = ep // (NW * CH)
    ei = ei.reshape(2, NW, n_ch, CH).transpose(1, 2, 0, 3)

    acc_rows = -(-(n_nodes + 1) // (NS * CH)) * (NS * CH)
    partial = _sc_scatter(n_nodes, d, ep, acc_rows)(features, ei)

    blk = 1000
    grid = n_nodes // blk
    out = pl.pallas_call(
        _combine_body,
        grid=(grid,),
        in_specs=[
            pl.BlockSpec((1, blk, d), lambda i: (0, i, 0)),
            pl.BlockSpec((1, blk, d), lambda i: (1, i, 0)),
            pl.BlockSpec((1, d), lambda i: (0, 0)),
        ],
        out_specs=pl.BlockSpec((blk, d), lambda i: (i, 0)),
        out_shape=jax.ShapeDtypeStruct((n_nodes, d), jnp.float32),
    )(partial, partial, W.reshape(1, d))
    return out


# speedup vs baseline: 1.1247x; 1.1247x over previous
import functools

import jax
import jax.numpy as jnp
from jax import lax
from jax.experimental import pallas as pl
from jax.experimental.pallas import tpu as pltpu
from jax.experimental.pallas import tpu_sc as plsc

NC = 2
NS = 16
L = 16
NW = NC * NS

CH = 128


def _sc_scatter(n_nodes, d, ep, acc_rows):
    e_per_tile = ep // NW
    n_ch = e_per_tile // CH
    nbuf = 2
    assert n_ch % nbuf == 0
    rows_per_tile = acc_rows // NS
    n_zero = rows_per_tile // CH

    mesh = plsc.VectorSubcoreMesh(core_axis_name="c", subcore_axis_name="s")

    @functools.partial(
        pl.kernel,
        mesh=mesh,
        out_type=jax.ShapeDtypeStruct((NC, acc_rows, d), jnp.float32),
        scratch_types=(
            [pltpu.VMEM((CH,), jnp.int32) for _ in range(4)]
            + [pltpu.VMEM((CH, d), jnp.float32) for _ in range(2)]
            + [pltpu.VMEM_SHARED((acc_rows, d), jnp.float32)]
            + [pltpu.SemaphoreType.DMA for _ in range(4)]
        ),
    )
    def k(feat_hbm, src_hbm, dst_hbm, out_hbm, src0, dst0, src1, dst1,
          rows0, rows1, acc_sh, g0, g1, s0, s1):
        cid = lax.axis_index("c")
        sid = lax.axis_index("s")
        wid = sid * NC + cid

        def zero_body(i, _):
            for j in range(d // L):
                rows0[i, pl.ds(j * L, L)] = jnp.zeros((L,), jnp.float32)
            return _
        lax.fori_loop(0, CH, zero_body, None)
        acc_base = sid * rows_per_tile
        for j in range(n_zero):
            pltpu.sync_copy(rows0, acc_sh.at[pl.ds(acc_base + j * CH, CH)])
        plsc.subcore_barrier()

        ebase = wid * e_per_tile

        def fetch(sref, dref, c):
            off = ebase + c * CH
            pltpu.sync_copy(src_hbm.at[pl.ds(off, CH)], sref)
            pltpu.sync_copy(dst_hbm.at[pl.ds(off, CH)], dref)

        def gstart(sref, rows, sem):
            pltpu.async_copy(feat_hbm.at[sref], rows, sem)

        def gwait(sref, rows, sem):
            pltpu.make_async_copy(feat_hbm.at[sref], rows, sem).wait()

        def sstart(dref, rows, sem):
            pltpu.async_copy(rows, acc_sh.at[dref], sem, add=True)

        def swait(dref, rows, sem):
            pltpu.make_async_copy(rows, acc_sh.at[dref], sem).wait()

        fetch(src0, dst0, 0)
        gstart(src0, rows0, g0)

        def edge_body(g, _):
            c0 = 2 * g
            fetch(src1, dst1, c0 + 1)
            gwait(src0, rows0, g0)
            sstart(dst0, rows0, s0)
            gstart(src1, rows1, g1)
            swait(dst0, rows0, s0)
            fetch(src0, dst0, jnp.minimum(c0 + 2, n_ch - 1))
            gstart(src0, rows0, g0)
            gwait(src1, rows1, g1)
            sstart(dst1, rows1, s1)
            swait(dst1, rows1, s1)
            return _
        lax.fori_loop(0, n_ch // 2, edge_body, None)
        gwait(src0, rows0, g0)
        plsc.subcore_barrier()

        pltpu.sync_copy(
            acc_sh.at[pl.ds(acc_base, rows_per_tile)],
            out_hbm.at[cid, pl.ds(acc_base, rows_per_tile)],
        )

    return k


def _combine_body(p0_ref, p1_ref, w_ref, o_ref):
    o_ref[...] = (p0_ref[0] + p1_ref[0]) * w_ref[...]


def kernel(features, edge_index, W):
    n_nodes, d = features.shape
    e = edge_index.shape[1]

    ep = -(-e // (2 * NW * CH)) * (2 * NW * CH)
    ei = edge_index
    if ep != e:
        pad = ep - e
        ei = jnp.concatenate(
            [ei, jnp.stack([jnp.zeros((pad,), jnp.int32),
                            jnp.full((pad,), n_nodes, jnp.int32)])], axis=1)

    acc_rows = -(-(n_nodes + 1) // (NS * CH)) * (NS * CH)
    partial = _sc_scatter(n_nodes, d, ep, acc_rows)(features, ei[0], ei[1])

    blk = 1000
    grid = n_nodes // blk
    out = pl.pallas_call(
        _combine_body,
        grid=(grid,),
        in_specs=[
            pl.BlockSpec((1, blk, d), lambda i: (0, i, 0)),
            pl.BlockSpec((1, blk, d), lambda i: (1, i, 0)),
            pl.BlockSpec((1, d), lambda i: (0, 0)),
        ],
        out_specs=pl.BlockSpec((blk, d), lambda i: (i, 0)),
        out_shape=jax.ShapeDtypeStruct((n_nodes, d), jnp.float32),
    )(partial, partial, W.reshape(1, d))
    return out
